# Initial kernel scaffold; baseline (speedup 1.0000x reference)
import functools

import jax
import jax.numpy as jnp
from jax import lax
from jax.experimental import pallas as pl
from jax.experimental.pallas import tpu as pltpu
from jax.experimental.pallas import tpu_sc as plsc

D = 128
E_PAD = 102400


def _mm_kernel(a_ref, w_ref, b_ref, o_ref):
    o_ref[...] = (
        jnp.dot(a_ref[...], w_ref[...], preferred_element_type=jnp.float32)
        + b_ref[...]
    )


def _mm_bias(a, w, bias, bm):
    n, k = a.shape
    grid = n // bm
    return pl.pallas_call(
        _mm_kernel,
        grid=(grid,),
        in_specs=[
            pl.BlockSpec((bm, k), lambda i: (i, 0)),
            pl.BlockSpec((k, D), lambda i: (0, 0)),
            pl.BlockSpec((1, D), lambda i: (0, 0)),
        ],
        out_specs=pl.BlockSpec((bm, D), lambda i: (i, 0)),
        out_shape=jax.ShapeDtypeStruct((n, D), jnp.float32),
    )(a, w, bias.reshape(1, D))


def _fuse_gather(t_lt, t_bv, t_b, bill_of, vote_lt, vote_bv):
    info = plsc.get_sparse_core_info()
    nc, ns = info.num_cores, info.num_subcores
    nw = nc * ns
    per_w = E_PAD // nw
    blk = 320
    nblk = per_w // blk

    mesh = plsc.VectorSubcoreMesh(core_axis_name="c", subcore_axis_name="s")

    @functools.partial(
        pl.kernel,
        mesh=mesh,
        out_type=jax.ShapeDtypeStruct((E_PAD, D), jnp.float32),
        scratch_types=[
            pltpu.VMEM((per_w,), jnp.int32),
            pltpu.VMEM((per_w,), jnp.int32),
            pltpu.VMEM((per_w,), jnp.int32),
            pltpu.VMEM((blk, D), jnp.float32),
            pltpu.SemaphoreType.DMA,
        ],
    )
    def k(tlt_hbm, tbv_hbm, tb_hbm, bof_hbm, vlt_hbm, vbv_hbm, out_hbm,
          ilt_v, ibv_v, ib_v, acc_v, sem):
        wid = lax.axis_index("s") * nc + lax.axis_index("c")
        base = wid * per_w
        pltpu.sync_copy(vlt_hbm.at[pl.ds(base, per_w)], ilt_v)
        pltpu.sync_copy(vbv_hbm.at[pl.ds(base, per_w)], ibv_v)
        pltpu.async_copy(bof_hbm.at[ibv_v], ib_v, sem).wait()

        def body(i, carry):
            off = i * blk
            s = pl.ds(off, blk)
            pltpu.async_copy(tlt_hbm.at[ilt_v.at[s]], acc_v, sem).wait()
            pltpu.async_copy(tbv_hbm.at[ibv_v.at[s]], acc_v, sem,
                             add=True).wait()
            pltpu.async_copy(tb_hbm.at[ib_v.at[s]], acc_v, sem,
                             add=True).wait()
            pltpu.sync_copy(acc_v, out_hbm.at[pl.ds(base + off, blk)])
            return carry

        lax.fori_loop(0, nblk, body, 0)

    return k(t_lt, t_bv, t_b, bill_of, vote_lt, vote_bv)


def kernel(h_bill, h_bill_version, h_legislator_term, h_committee, h_donor,
           h_lobby_firm, h_topic, vote_time, vote_lt, vote_bv, topic_ix,
           ei_is_version, ei_prior, ei_read, ei_member, ei_donated,
           ei_lobbied, W_fuse, b_fuse):
    nb = h_bill.shape[0]
    nbv = h_bill_version.shape[0]
    nlt = h_legislator_term.shape[0]
    bill_of = ei_is_version[1]

    def smean(src, idx, n):
        s = jax.ops.segment_sum(src, idx, num_segments=n)
        c = jax.ops.segment_sum(jnp.ones((src.shape[0], 1), src.dtype), idx,
                                num_segments=n)
        return s / jnp.maximum(c, 1.0)

    pv_pool = smean(h_bill_version[ei_prior[0]], ei_prior[1], nbv)
    bv_pool = smean(h_committee[ei_read[1]], ei_read[0], nbv)
    bill_comm = smean(bv_pool, bill_of, nb)
    lt_comm = smean(h_committee[ei_member[1]], ei_member[0], nlt)
    lt_don = smean(h_donor[ei_donated[0]], ei_donated[1], nlt)
    lt_lob = smean(h_lobby_firm[ei_lobbied[0]], ei_lobbied[1], nlt)

    W = W_fuse
    W1, W2, W3, W4, W5, W6 = (W[i * D:(i + 1) * D] for i in range(6))

    a_lt = jnp.concatenate([h_legislator_term, lt_comm, lt_don + lt_lob], 1)
    w_lt = jnp.concatenate([W1, W4, W5], 0)
    t_lt = _mm_bias(a_lt, w_lt, b_fuse, bm=500)

    t_bv = _mm_bias(pv_pool, W2, jnp.zeros((D,), jnp.float32), bm=500)

    tix = jnp.clip(topic_ix, 0, None)
    a_b = jnp.concatenate([bill_comm, h_topic[tix]], 1)
    w_b = jnp.concatenate([W3, W6], 0)
    t_b = _mm_bias(a_b, w_b, jnp.zeros((D,), jnp.float32), bm=500)

    pad = E_PAD - vote_lt.shape[0]
    vlt = jnp.concatenate([vote_lt, jnp.zeros((pad,), jnp.int32)])
    vbv = jnp.concatenate([vote_bv, jnp.zeros((pad,), jnp.int32)])
    out = _fuse_gather(t_lt, t_bv, t_b, bill_of, vlt, vbv)
    return out[: vote_lt.shape[0]]


# speedup vs baseline: 1.6618x; 1.6618x over previous
import functools

import jax
import jax.numpy as jnp
from jax import lax
from jax.experimental import pallas as pl
from jax.experimental.pallas import tpu as pltpu
from jax.experimental.pallas import tpu_sc as plsc

D = 128
E_PAD = 102400


def _mm_kernel(a_ref, w_ref, b_ref, o_ref):
    o_ref[...] = (
        jnp.dot(a_ref[...], w_ref[...], preferred_element_type=jnp.float32)
        + b_ref[...]
    )


def _mm_bias(a, w, bias, bm):
    n, k = a.shape
    grid = n // bm
    return pl.pallas_call(
        _mm_kernel,
        grid=(grid,),
        in_specs=[
            pl.BlockSpec((bm, k), lambda i: (i, 0)),
            pl.BlockSpec((k, D), lambda i: (0, 0)),
            pl.BlockSpec((1, D), lambda i: (0, 0)),
        ],
        out_specs=pl.BlockSpec((bm, D), lambda i: (i, 0)),
        out_shape=jax.ShapeDtypeStruct((n, D), jnp.float32),
    )(a, w, bias.reshape(1, D))


def _fuse_gather(t_lt, t_bv, t_b, bill_of, vote_lt, vote_bv):
    info = plsc.get_sparse_core_info()
    nc, ns = info.num_cores, info.num_subcores
    nw = nc * ns
    per_w = E_PAD // nw
    blk = 320
    nblk = per_w // blk

    mesh = plsc.VectorSubcoreMesh(core_axis_name="c", subcore_axis_name="s")

    @functools.partial(
        pl.kernel,
        mesh=mesh,
        out_type=jax.ShapeDtypeStruct((E_PAD, D), jnp.float32),
        scratch_types=[
            pltpu.VMEM((per_w,), jnp.int32),
            pltpu.VMEM((per_w,), jnp.int32),
            pltpu.VMEM((per_w,), jnp.int32),
            pltpu.VMEM((blk, D), jnp.float32),
            pltpu.SemaphoreType.DMA,
        ],
    )
    def k(tlt_hbm, tbv_hbm, tb_hbm, bof_hbm, vlt_hbm, vbv_hbm, out_hbm,
          ilt_v, ibv_v, ib_v, acc_v, sem):
        wid = lax.axis_index("s") * nc + lax.axis_index("c")
        base = wid * per_w
        pltpu.sync_copy(vlt_hbm.at[pl.ds(base, per_w)], ilt_v)
        pltpu.sync_copy(vbv_hbm.at[pl.ds(base, per_w)], ibv_v)
        pltpu.async_copy(bof_hbm.at[ibv_v], ib_v, sem).wait()

        def body(i, carry):
            off = i * blk
            s = pl.ds(off, blk)
            pltpu.async_copy(tlt_hbm.at[ilt_v.at[s]], acc_v, sem).wait()
            pltpu.async_copy(tbv_hbm.at[ibv_v.at[s]], acc_v, sem,
                             add=True).wait()
            pltpu.async_copy(tb_hbm.at[ib_v.at[s]], acc_v, sem,
                             add=True).wait()
            pltpu.sync_copy(acc_v, out_hbm.at[pl.ds(base + off, blk)])
            return carry

        lax.fori_loop(0, nblk, body, 0)

    return k(t_lt, t_bv, t_b, bill_of, vote_lt, vote_bv)


def kernel(h_bill, h_bill_version, h_legislator_term, h_committee, h_donor,
           h_lobby_firm, h_topic, vote_time, vote_lt, vote_bv, topic_ix,
           ei_is_version, ei_prior, ei_read, ei_member, ei_donated,
           ei_lobbied, W_fuse, b_fuse):
    nb = h_bill.shape[0]
    nbv = h_bill_version.shape[0]
    nlt = h_legislator_term.shape[0]
    bill_of = ei_is_version[1]

    def smean(src, idx, n):
        s = jax.ops.segment_sum(src, idx, num_segments=n)
        c = jax.ops.segment_sum(jnp.ones((src.shape[0], 1), src.dtype), idx,
                                num_segments=n)
        return s / jnp.maximum(c, 1.0)

    pv_pool = smean(h_bill_version[ei_prior[0]], ei_prior[1], nbv)
    bv_pool = smean(h_committee[ei_read[1]], ei_read[0], nbv)
    bill_comm = smean(bv_pool, bill_of, nb)
    lt_comm = smean(h_committee[ei_member[1]], ei_member[0], nlt)
    lt_don = smean(h_donor[ei_donated[0]], ei_donated[1], nlt)
    lt_lob = smean(h_lobby_firm[ei_lobbied[0]], ei_lobbied[1], nlt)

    W = W_fuse
    W1, W2, W3, W4, W5, W6 = (W[i * D:(i + 1) * D] for i in range(6))

    a_lt = jnp.concatenate([h_legislator_term, lt_comm, lt_don + lt_lob], 1)
    w_lt = jnp.concatenate([W1, W4, W5], 0)
    t_lt = _mm_bias(a_lt, w_lt, b_fuse, bm=1000)

    t_bv = _mm_bias(pv_pool, W2, jnp.zeros((D,), jnp.float32), bm=1000)

    tix = jnp.clip(topic_ix, 0, None)
    a_b = jnp.concatenate([bill_comm, h_topic[tix]], 1)
    w_b = jnp.concatenate([W3, W6], 0)
    t_b = _mm_bias(a_b, w_b, jnp.zeros((D,), jnp.float32), bm=1000)

    pad = E_PAD - vote_lt.shape[0]
    vlt = jnp.concatenate([vote_lt, jnp.zeros((pad,), jnp.int32)])
    vbv = jnp.concatenate([vote_bv, jnp.zeros((pad,), jnp.int32)])
    out = _fuse_gather(t_lt, t_bv, t_b, bill_of, vlt, vbv)
    return out[: vote_lt.shape[0]]
